# baseline (speedup 1.0000x reference)
import functools

import jax
import jax.numpy as jnp
from jax import lax
from jax.experimental import pallas as pl
from jax.experimental.pallas import tpu as pltpu
from jax.experimental.pallas import tpu_sc as plsc

_B, _L, _DN, _D = 4096, 200, 128, 128
_BL = _B * _L
_NC, _NS = 2, 16
_NW = _NC * _NS
_CH = 128
_ITEM_BITS = 17
_ITEM_MASK = (1 << _ITEM_BITS) - 1
_VCATE = 1002
_NA = 4
_NB = 2
_SLICES = (512, 1536, 2048)


def _when(cond, fn):
  if isinstance(cond, bool):
    if cond:
      fn()
  else:
    pl.when(cond)(fn)


def _make_gather_sum(sbl):
  cpw = sbl // _NW // _CH
  total_r = sbl // _CH
  nload = -(-(cpw + 8) // 8) * 8

  def body(pk_hbm, itab_hbm, ctab_hbm, out_hbm,
           pk_v, islot, cslot,
           a0, a1, a2, a3, b0, b1, ctab_sh,
           ga0, ga1, ga2, ga3, gb0, gb1,
           gs0, gs1, gs2, gs3):
    A = (a0, a1, a2, a3)
    Bb = (b0, b1)
    GA = (ga0, ga1, ga2, ga3)
    GB = (gb0, gb1)
    GS = (gs0, gs1, gs2, gs3)

    wid = lax.axis_index("s") * _NC + lax.axis_index("c")
    base_r = wid * cpw
    align = jnp.minimum((base_r // 8) * 8, total_r - nload)
    lead = base_r - align
    @pl.when(lax.axis_index("s") == 0)
    def _():
      pltpu.sync_copy(ctab_hbm, ctab_sh)
    pltpu.sync_copy(pk_hbm.at[pl.ds(align, nload)], pk_v)
    plsc.subcore_barrier()

    def unpack(j, s):
      for t in range(_CH // 16):
        v = pk_v[lead + j, pl.ds(t * 16, 16)]
        islot[s, pl.ds(t * 16, 16)] = v & _ITEM_MASK
        cslot[s, pl.ds(t * 16, 16)] = lax.shift_right_logical(v, _ITEM_BITS)

    def issue_item(s):
      pltpu.async_copy(itab_hbm.at[islot.at[s]], A[s], GA[s])

    def issue_cate(s, bslot):
      pltpu.async_copy(ctab_sh.at[cslot.at[s]], Bb[bslot], GB[bslot])

    for c in range(_NA - 1):
      unpack(c, c)
      issue_item(c)
      if c < _NB:
        issue_cate(c, c)

    def substep(c, k):
      a = k % _NA
      b = k % _NB
      pltpu.make_async_copy(itab_hbm.at[islot.at[a]], A[a], GA[a]).wait()
      pltpu.make_async_copy(ctab_sh.at[cslot.at[a]], Bb[b], GB[b]).wait()
      _when(c + _NA - 1 < cpw,
            lambda: unpack(c + _NA - 1, (k + _NA - 1) % _NA))
      def addrows(r, carry):
        for rr in range(4):
          for t in range(_D // 16):
            plsc.addupdate(A[a].at[r * 4 + rr, pl.ds(t * 16, 16)],
                           Bb[b][r * 4 + rr, pl.ds(t * 16, 16)])
        return carry
      lax.fori_loop(0, _CH // 4, addrows, 0)
      pltpu.async_copy(A[a], out_hbm.at[pl.ds((base_r + c) * _CH, _CH)], GS[a])
      _when(c + _NB < cpw, lambda: issue_cate((k + _NB) % _NA, b))
      _when(c >= 1,
            lambda: pltpu.make_async_copy(
                A[(k + _NA - 1) % _NA], out_hbm.at[pl.ds(0, _CH)],
                GS[(k + _NA - 1) % _NA]).wait())
      _when(c + _NA - 1 < cpw, lambda: issue_item((k + _NA - 1) % _NA))

    def round_(r, carry):
      for k in range(_NA):
        substep(r * _NA + k, k)
      return carry

    rounds = cpw // _NA
    lax.fori_loop(0, rounds, round_, 0)
    for c in range(rounds * _NA, cpw):
      substep(c, c % _NA)
    pltpu.make_async_copy(A[(cpw - 1) % _NA], out_hbm.at[pl.ds(0, _CH)],
                          GS[(cpw - 1) % _NA]).wait()

  return functools.partial(
      pl.kernel,
      out_type=jax.ShapeDtypeStruct((sbl, _D), jnp.float32),
      mesh=plsc.VectorSubcoreMesh(core_axis_name="c", subcore_axis_name="s"),
      scratch_types=[
          pltpu.VMEM((nload, _CH), jnp.int32),
          pltpu.VMEM((_NA, _CH), jnp.int32),
          pltpu.VMEM((_NA, _CH), jnp.int32),
          pltpu.VMEM((_CH, _D), jnp.float32),
          pltpu.VMEM((_CH, _D), jnp.float32),
          pltpu.VMEM((_CH, _D), jnp.float32),
          pltpu.VMEM((_CH, _D), jnp.float32),
          pltpu.VMEM((_CH, _D), jnp.float32),
          pltpu.VMEM((_CH, _D), jnp.float32),
          pltpu.VMEM_SHARED((_VCATE, _D), jnp.float32),
      ] + [pltpu.SemaphoreType.DMA] * 10,
  )(body)


_gather_sums = {sb: _make_gather_sum(sb * _L) for sb in set(_SLICES)}

_BT = 64


def _fuse_first_body(price_ref, partial_ref, len_ref, w_ref, b_ref, pos_ref,
                     seq_ref, mask_ref):
  acc = jnp.dot(price_ref[...], w_ref[...],
                preferred_element_type=jnp.float32)
  acc = acc + partial_ref[...] + b_ref[...]
  acc = acc.reshape(_BT, _L, _D) + pos_ref[...][None, :, :]
  seq_ref[...] = acc.reshape(_BT * _L, _D)
  lens = len_ref[...]
  mask_ref[...] = lax.broadcasted_iota(jnp.int32, (_BT, _L), 1) < lens


def _fuse_next_body(seq_in, mask_in, price_ref, partial_ref, len_ref, w_ref,
                    b_ref, pos_ref, seq_ref, mask_ref):
  del seq_in, mask_in
  _fuse_first_body(price_ref, partial_ref, len_ref, w_ref, b_ref, pos_ref,
                   seq_ref, mask_ref)


def _make_fuse(start_b, sb, first):
  off = start_b // _BT
  sgrid = sb // _BT
  common_in = [
      pl.BlockSpec((_BT * _L, _D), lambda i: (off + i, 0)),
      pl.BlockSpec((_BT * _L, _D), lambda i: (i, 0)),
      pl.BlockSpec((_BT, 1), lambda i: (off + i, 0)),
      pl.BlockSpec((_DN, _D), lambda i: (0, 0)),
      pl.BlockSpec((1, _D), lambda i: (0, 0)),
      pl.BlockSpec((_L, _D), lambda i: (0, 0)),
  ]
  out_specs = [
      pl.BlockSpec((_BT * _L, _D), lambda i: (off + i, 0)),
      pl.BlockSpec((_BT, _L), lambda i: (off + i, 0)),
  ]
  out_shape = [
      jax.ShapeDtypeStruct((_BL, _D), jnp.float32),
      jax.ShapeDtypeStruct((_B, _L), jnp.bool_),
  ]
  params = pltpu.CompilerParams(dimension_semantics=("parallel",))
  if first:
    return pl.pallas_call(
        _fuse_first_body, grid=(sgrid,), in_specs=common_in,
        out_specs=out_specs, out_shape=out_shape, compiler_params=params)
  return pl.pallas_call(
      _fuse_next_body, grid=(sgrid,),
      in_specs=[pl.BlockSpec(memory_space=pl.ANY),
                pl.BlockSpec(memory_space=pl.ANY)] + common_in,
      out_specs=out_specs, out_shape=out_shape,
      input_output_aliases={0: 0, 1: 1}, compiler_params=params)


_starts = [sum(_SLICES[:i]) for i in range(len(_SLICES))]
_fuses = [_make_fuse(_starts[i], _SLICES[i], i == 0)
          for i in range(len(_SLICES))]


def kernel(item_id, cate_id, price, length, item_table, cate_table, W, b,
           pos_table):
  packed = item_id.astype(jnp.int32) | (cate_id.astype(jnp.int32) << _ITEM_BITS)
  packed = packed.reshape(_BL // _CH, _CH)
  price_f = price.reshape(_BL, _DN)
  b2 = b.reshape(1, _D)
  partials = []
  for i, sb in enumerate(_SLICES):
    r0 = _starts[i] * _L // _CH
    r1 = r0 + sb * _L // _CH
    partials.append(
        _gather_sums[sb](packed[r0:r1], item_table, cate_table))
  seq, mask = _fuses[0](price_f, partials[0], length, W, b2, pos_table)
  for i in range(1, len(_SLICES)):
    seq, mask = _fuses[i](seq, mask, price_f, partials[i], length, W, b2,
                          pos_table)
  return seq.reshape(_B, _L, _D), mask


# speedup vs baseline: 1.0016x; 1.0016x over previous
import functools

import jax
import jax.numpy as jnp
from jax import lax
from jax.experimental import pallas as pl
from jax.experimental.pallas import tpu as pltpu
from jax.experimental.pallas import tpu_sc as plsc

_B, _L, _DN, _D = 4096, 200, 128, 128
_BL = _B * _L
_NC, _NS = 2, 16
_NW = _NC * _NS
_CH = 128
_ITEM_BITS = 17
_ITEM_MASK = (1 << _ITEM_BITS) - 1
_VCATE = 1002
_NA = 4
_NB = 2
_SLICES = (2048, 2048)


def _when(cond, fn):
  if isinstance(cond, bool):
    if cond:
      fn()
  else:
    pl.when(cond)(fn)


def _make_gather_sum(sbl):
  cpw = sbl // _NW // _CH
  total_r = sbl // _CH
  nload = -(-(cpw + 8) // 8) * 8

  def body(pk_hbm, itab_hbm, ctab_hbm, out_hbm,
           pk_v, islot, cslot,
           a0, a1, a2, a3, b0, b1, ctab_sh,
           ga0, ga1, ga2, ga3, gb0, gb1,
           gs0, gs1, gs2, gs3):
    A = (a0, a1, a2, a3)
    Bb = (b0, b1)
    GA = (ga0, ga1, ga2, ga3)
    GB = (gb0, gb1)
    GS = (gs0, gs1, gs2, gs3)

    wid = lax.axis_index("s") * _NC + lax.axis_index("c")
    base_r = wid * cpw
    align = jnp.minimum((base_r // 8) * 8, total_r - nload)
    lead = base_r - align
    @pl.when(lax.axis_index("s") == 0)
    def _():
      pltpu.sync_copy(ctab_hbm, ctab_sh)
    pltpu.sync_copy(pk_hbm.at[pl.ds(align, nload)], pk_v)
    plsc.subcore_barrier()

    def unpack(j, s):
      for t in range(_CH // 16):
        v = pk_v[lead + j, pl.ds(t * 16, 16)]
        islot[s, pl.ds(t * 16, 16)] = v & _ITEM_MASK
        cslot[s, pl.ds(t * 16, 16)] = lax.shift_right_logical(v, _ITEM_BITS)

    def issue_item(s):
      pltpu.async_copy(itab_hbm.at[islot.at[s]], A[s], GA[s])

    def issue_cate(s, bslot):
      pltpu.async_copy(ctab_sh.at[cslot.at[s]], Bb[bslot], GB[bslot])

    for c in range(_NA - 1):
      unpack(c, c)
      issue_item(c)
      if c < _NB:
        issue_cate(c, c)

    def substep(c, k):
      a = k % _NA
      b = k % _NB
      pltpu.make_async_copy(itab_hbm.at[islot.at[a]], A[a], GA[a]).wait()
      pltpu.make_async_copy(ctab_sh.at[cslot.at[a]], Bb[b], GB[b]).wait()
      _when(c + _NA - 1 < cpw,
            lambda: unpack(c + _NA - 1, (k + _NA - 1) % _NA))
      def addrows(r, carry):
        for rr in range(4):
          for t in range(_D // 16):
            plsc.addupdate(A[a].at[r * 4 + rr, pl.ds(t * 16, 16)],
                           Bb[b][r * 4 + rr, pl.ds(t * 16, 16)])
        return carry
      lax.fori_loop(0, _CH // 4, addrows, 0)
      pltpu.async_copy(A[a], out_hbm.at[pl.ds((base_r + c) * _CH, _CH)], GS[a])
      _when(c + _NB < cpw, lambda: issue_cate((k + _NB) % _NA, b))
      _when(c >= 1,
            lambda: pltpu.make_async_copy(
                A[(k + _NA - 1) % _NA], out_hbm.at[pl.ds(0, _CH)],
                GS[(k + _NA - 1) % _NA]).wait())
      _when(c + _NA - 1 < cpw, lambda: issue_item((k + _NA - 1) % _NA))

    def round_(r, carry):
      for k in range(_NA):
        substep(r * _NA + k, k)
      return carry

    rounds = cpw // _NA
    lax.fori_loop(0, rounds, round_, 0)
    for c in range(rounds * _NA, cpw):
      substep(c, c % _NA)
    pltpu.make_async_copy(A[(cpw - 1) % _NA], out_hbm.at[pl.ds(0, _CH)],
                          GS[(cpw - 1) % _NA]).wait()

  return functools.partial(
      pl.kernel,
      out_type=jax.ShapeDtypeStruct((sbl, _D), jnp.float32),
      mesh=plsc.VectorSubcoreMesh(core_axis_name="c", subcore_axis_name="s"),
      scratch_types=[
          pltpu.VMEM((nload, _CH), jnp.int32),
          pltpu.VMEM((_NA, _CH), jnp.int32),
          pltpu.VMEM((_NA, _CH), jnp.int32),
          pltpu.VMEM((_CH, _D), jnp.float32),
          pltpu.VMEM((_CH, _D), jnp.float32),
          pltpu.VMEM((_CH, _D), jnp.float32),
          pltpu.VMEM((_CH, _D), jnp.float32),
          pltpu.VMEM((_CH, _D), jnp.float32),
          pltpu.VMEM((_CH, _D), jnp.float32),
          pltpu.VMEM_SHARED((_VCATE, _D), jnp.float32),
      ] + [pltpu.SemaphoreType.DMA] * 10,
  )(body)


_gather_sums = {sb: _make_gather_sum(sb * _L) for sb in set(_SLICES)}

_BT = 64


def _fuse_first_body(price_ref, partial_ref, len_ref, w_ref, b_ref, pos_ref,
                     seq_ref, mask_ref):
  acc = jnp.dot(price_ref[...], w_ref[...],
                preferred_element_type=jnp.float32)
  acc = acc + partial_ref[...] + b_ref[...]
  acc = acc.reshape(_BT, _L, _D) + pos_ref[...][None, :, :]
  seq_ref[...] = acc.reshape(_BT * _L, _D)
  lens = len_ref[...]
  mask_ref[...] = lax.broadcasted_iota(jnp.int32, (_BT, _L), 1) < lens


def _fuse_next_body(seq_in, mask_in, price_ref, partial_ref, len_ref, w_ref,
                    b_ref, pos_ref, seq_ref, mask_ref):
  del seq_in, mask_in
  _fuse_first_body(price_ref, partial_ref, len_ref, w_ref, b_ref, pos_ref,
                   seq_ref, mask_ref)


def _make_fuse(start_b, sb, first):
  off = start_b // _BT
  sgrid = sb // _BT
  common_in = [
      pl.BlockSpec((_BT * _L, _D), lambda i: (off + i, 0)),
      pl.BlockSpec((_BT * _L, _D), lambda i: (i, 0)),
      pl.BlockSpec((_BT, 1), lambda i: (off + i, 0)),
      pl.BlockSpec((_DN, _D), lambda i: (0, 0)),
      pl.BlockSpec((1, _D), lambda i: (0, 0)),
      pl.BlockSpec((_L, _D), lambda i: (0, 0)),
  ]
  out_specs = [
      pl.BlockSpec((_BT * _L, _D), lambda i: (off + i, 0)),
      pl.BlockSpec((_BT, _L), lambda i: (off + i, 0)),
  ]
  out_shape = [
      jax.ShapeDtypeStruct((_BL, _D), jnp.float32),
      jax.ShapeDtypeStruct((_B, _L), jnp.bool_),
  ]
  params = pltpu.CompilerParams(dimension_semantics=("parallel",))
  if first:
    return pl.pallas_call(
        _fuse_first_body, grid=(sgrid,), in_specs=common_in,
        out_specs=out_specs, out_shape=out_shape, compiler_params=params)
  return pl.pallas_call(
      _fuse_next_body, grid=(sgrid,),
      in_specs=[pl.BlockSpec(memory_space=pl.ANY),
                pl.BlockSpec(memory_space=pl.ANY)] + common_in,
      out_specs=out_specs, out_shape=out_shape,
      input_output_aliases={0: 0, 1: 1}, compiler_params=params)


_starts = [sum(_SLICES[:i]) for i in range(len(_SLICES))]
_fuses = [_make_fuse(_starts[i], _SLICES[i], i == 0)
          for i in range(len(_SLICES))]


def kernel(item_id, cate_id, price, length, item_table, cate_table, W, b,
           pos_table):
  packed = item_id.astype(jnp.int32) | (cate_id.astype(jnp.int32) << _ITEM_BITS)
  packed = packed.reshape(_BL // _CH, _CH)
  price_f = price.reshape(_BL, _DN)
  b2 = b.reshape(1, _D)
  partials = []
  for i, sb in enumerate(_SLICES):
    r0 = _starts[i] * _L // _CH
    r1 = r0 + sb * _L // _CH
    partials.append(
        _gather_sums[sb](packed[r0:r1], item_table, cate_table))
  seq, mask = _fuses[0](price_f, partials[0], length, W, b2, pos_table)
  for i in range(1, len(_SLICES)):
    seq, mask = _fuses[i](seq, mask, price_f, partials[i], length, W, b2,
                          pos_table)
  return seq.reshape(_B, _L, _D), mask
